# baseline (speedup 1.0000x reference)
import jax
import jax.numpy as jnp
from jax import lax
from jax.experimental import pallas as pl
from jax.experimental.pallas import tpu as pltpu
from jax.experimental.pallas import tpu_sc as plsc

N = 10000
E = 320000
D = 128
C = 128
NW = 32
ROWS_W = 84
GROUP = 12
NCHUNK_PAD = 2696
NSUB = 16
ROWS_PER_TILE = N // NSUB
LANES = 16
ZROWS = 125
NBUF = 3
NSET = 4


def _finish_body(p0_ref, p1_ref, w_ref, b_ref, o_ref):
    o_ref[...] = jnp.dot(
        p0_ref[...] + p1_ref[...], w_ref[...],
        preferred_element_type=jnp.float32,
        precision=lax.Precision.HIGHEST,
    ) + b_ref[...]


def _spmm_body(support_hbm, packed_hbm, out_hbm, *sc):
    idx_b = sc[0:4]
    isems = sc[4:8]
    rows = sc[8:11]
    gsems = sc[11:14]
    ssems = sc[14:17]
    acc = sc[17]

    c = lax.axis_index("c")
    s = lax.axis_index("s")
    wid = c * NSUB + s
    base_chunk = wid * ROWS_W

    def start_idx(t, j):
        i = j % NSET
        pltpu.async_copy(packed_hbm.at[t], idx_b[i], isems[i])

    def wait_idx(j):
        i = j % NSET
        pltpu.make_async_copy(packed_hbm.at[0], idx_b[i], isems[i]).wait()

    def gather_start(t_j, b_j):
        pltpu.async_copy(support_hbm.at[idx_b[t_j % NSET].at[0]],
                         rows[b_j % NBUF], gsems[b_j % NBUF])

    def gather_wait(t_j, b_j):
        pltpu.make_async_copy(support_hbm.at[idx_b[t_j % NSET].at[0]],
                              rows[b_j % NBUF], gsems[b_j % NBUF]).wait()

    def scatter_start(t_j, b_j):
        pltpu.async_copy(rows[b_j % NBUF], acc.at[idx_b[t_j % NSET].at[1]],
                         ssems[b_j % NBUF], add=True)

    def scatter_wait(t_j, b_j):
        pltpu.make_async_copy(rows[b_j % NBUF], acc.at[idx_b[t_j % NSET].at[1]],
                              ssems[b_j % NBUF]).wait()

    for j in range(NSET):
        start_idx(base_chunk + j, j)

    zeros16 = jnp.zeros((LANES,), jnp.float32)

    for rbuf in rows:
        @pl.loop(0, C)
        def _zero_row(i):
            row = rbuf.at[i]
            for g in range(D // LANES):
                row[pl.ds(g * LANES, LANES)] = zeros16

    base_row = s * ROWS_PER_TILE
    for k in range(ROWS_PER_TILE // ZROWS):
        pltpu.async_copy(rows[0].at[pl.ds(0, ZROWS)],
                         acc.at[pl.ds(base_row + k * ZROWS, ZROWS)], gsems[0])
    for k in range(ROWS_PER_TILE // ZROWS):
        pltpu.make_async_copy(rows[0].at[pl.ds(0, ZROWS)],
                              acc.at[pl.ds(0, ZROWS)], gsems[0]).wait()

    plsc.subcore_barrier()

    def scale(t_j, b_j):
        wrow = idx_b[t_j % NSET].at[2]
        rbuf = rows[b_j % NBUF]

        @pl.loop(0, C, unroll=4)
        def _scale(e):
            wsplat = plsc.bitcast(
                plsc.load_gather(wrow, [jnp.full((LANES,), e, jnp.int32)]),
                jnp.float32)
            row = rbuf.at[e]
            for g in range(D // LANES):
                sl = pl.ds(g * LANES, LANES)
                row[sl] = row[sl] * wsplat

    wait_idx(0)
    wait_idx(2)
    wait_idx(3)
    pltpu.async_copy(rows[1], acc.at[idx_b[2].at[1]], ssems[1], add=True)
    pltpu.async_copy(rows[2], acc.at[idx_b[3].at[1]], ssems[2], add=True)
    gather_start(0, 0)

    def chunk_step(t, j, first_group):
        scatter_wait(j - 2, j - 2)
        if not (first_group and j < 2):
            start_idx(t + 2, j + 2)
        if not (first_group and j in (1, 2)):
            wait_idx(j + 1)
        gather_start(j + 1, j + 1)
        gather_wait(j, j)
        scale(j, j)
        scatter_start(j, j)

    for j in range(GROUP):
        chunk_step(base_chunk + j, j, True)

    @pl.loop(GROUP, ROWS_W, step=GROUP)
    def _group(p):
        for j in range(GROUP):
            chunk_step(base_chunk + p + j, j, False)

    scatter_wait(ROWS_W - 2, ROWS_W - 2)
    scatter_wait(ROWS_W - 1, ROWS_W - 1)
    gather_wait(ROWS_W, ROWS_W)
    wait_idx(ROWS_W + 1)

    plsc.subcore_barrier()

    pltpu.sync_copy(acc.at[pl.ds(base_row, ROWS_PER_TILE)],
                    out_hbm.at[c, pl.ds(base_row, ROWS_PER_TILE)])


_spmm = pl.kernel(
    _spmm_body,
    out_type=jax.ShapeDtypeStruct((2, N, D), jnp.float32),
    mesh=plsc.VectorSubcoreMesh(core_axis_name="c", subcore_axis_name="s"),
    compiler_params=pltpu.CompilerParams(
        use_tc_tiling_on_sc=False, needs_layout_passes=False),
    scratch_types=(
        [pltpu.VMEM((3, C), jnp.int32) for _ in range(NSET)]
        + [pltpu.SemaphoreType.DMA for _ in range(NSET)]
        + [pltpu.VMEM((C, D), jnp.float32) for _ in range(NBUF)]
        + [pltpu.SemaphoreType.DMA for _ in range(NBUF)]
        + [pltpu.SemaphoreType.DMA for _ in range(NBUF)]
        + [pltpu.VMEM_SHARED((N, D), jnp.float32)]
    ),
)


def kernel(x, edge_index, edge_weight, W, b):
    npad = NCHUNK_PAD * C - E
    pad_idx = jnp.arange(npad, dtype=jnp.int32) % N
    src = jnp.concatenate([edge_index[0], pad_idx]).reshape(NCHUNK_PAD, 1, C)
    dst = jnp.concatenate([edge_index[1], pad_idx]).reshape(NCHUNK_PAD, 1, C)
    ewi = jax.lax.bitcast_convert_type(
        jnp.concatenate([edge_weight, jnp.zeros((npad,), jnp.float32)]),
        jnp.int32).reshape(NCHUNK_PAD, 1, C)
    packed = jnp.concatenate([src, dst, ewi], axis=1)

    partials = _spmm(x, packed)

    RB = 1000
    out = pl.pallas_call(
        _finish_body,
        grid=(N // RB,),
        in_specs=[
            pl.BlockSpec((RB, D), lambda i: (i, 0)),
            pl.BlockSpec((RB, D), lambda i: (i, 0)),
            pl.BlockSpec((D, D), lambda i: (0, 0)),
            pl.BlockSpec((1, D), lambda i: (0, 0)),
        ],
        out_specs=pl.BlockSpec((RB, D), lambda i: (i, 0)),
        out_shape=jax.ShapeDtypeStruct((N, D), jnp.float32),
    )(partials[0], partials[1], W, b.reshape(1, D))
    return out


# speedup vs baseline: 1.0303x; 1.0303x over previous
import jax
import jax.numpy as jnp
from jax import lax
from jax.experimental import pallas as pl
from jax.experimental.pallas import tpu as pltpu
from jax.experimental.pallas import tpu_sc as plsc

N = 10000
E = 320000
D = 128
C = 128
NW = 32
ROWS_W = 84
GROUP = 12
NCHUNK_PAD = 2696
NSUB = 16
ROWS_PER_TILE = N // NSUB
LANES = 16
ZROWS = 125
NBUF = 3
NSET = 4


def _finish_body(p0_ref, p1_ref, w_ref, b_ref, o_ref):
    o_ref[...] = jnp.dot(
        p0_ref[...] + p1_ref[...], w_ref[...],
        preferred_element_type=jnp.float32,
        precision=lax.Precision.HIGHEST,
    ) + b_ref[...]


def _spmm_body(support_hbm, src_hbm, dst_hbm, ew_hbm, out_hbm, *sc):
    src_b = sc[0:4]
    dst_b = sc[4:8]
    ew_b = sc[8:12]
    isems = sc[12:16]
    rows = sc[16:19]
    gsems = sc[19:22]
    ssems = sc[22:25]
    acc = sc[25]

    c = lax.axis_index("c")
    s = lax.axis_index("s")
    wid = c * NSUB + s
    base_chunk = wid * ROWS_W

    def start_idx(t, j):
        i = j % NSET
        pltpu.async_copy(src_hbm.at[t], src_b[i], isems[i])
        pltpu.async_copy(dst_hbm.at[t], dst_b[i], isems[i])
        pltpu.async_copy(ew_hbm.at[t], ew_b[i], isems[i])

    def wait_idx(j):
        i = j % NSET
        pltpu.make_async_copy(src_hbm.at[0], src_b[i], isems[i]).wait()
        pltpu.make_async_copy(dst_hbm.at[0], dst_b[i], isems[i]).wait()
        pltpu.make_async_copy(ew_hbm.at[0], ew_b[i], isems[i]).wait()

    def gather_start(t_j, b_j):
        pltpu.async_copy(support_hbm.at[src_b[t_j % NSET]],
                         rows[b_j % NBUF], gsems[b_j % NBUF])

    def gather_wait(t_j, b_j):
        pltpu.make_async_copy(support_hbm.at[src_b[t_j % NSET]],
                              rows[b_j % NBUF], gsems[b_j % NBUF]).wait()

    def scatter_start(t_j, b_j):
        pltpu.async_copy(rows[b_j % NBUF], acc.at[dst_b[t_j % NSET]],
                         ssems[b_j % NBUF], add=True)

    def scatter_wait(t_j, b_j):
        pltpu.make_async_copy(rows[b_j % NBUF], acc.at[dst_b[t_j % NSET]],
                              ssems[b_j % NBUF]).wait()

    for j in range(NSET):
        start_idx(base_chunk + j, j)

    zeros16 = jnp.zeros((LANES,), jnp.float32)

    for rbuf in rows:
        @pl.loop(0, C)
        def _zero_row(i):
            row = rbuf.at[i]
            for g in range(D // LANES):
                row[pl.ds(g * LANES, LANES)] = zeros16

    base_row = s * ROWS_PER_TILE
    for k in range(ROWS_PER_TILE // ZROWS):
        pltpu.async_copy(rows[0].at[pl.ds(0, ZROWS)],
                         acc.at[pl.ds(base_row + k * ZROWS, ZROWS)], gsems[0])
    for k in range(ROWS_PER_TILE // ZROWS):
        pltpu.make_async_copy(rows[0].at[pl.ds(0, ZROWS)],
                              acc.at[pl.ds(0, ZROWS)], gsems[0]).wait()

    plsc.subcore_barrier()

    def scale(t_j, b_j):
        wrow = ew_b[t_j % NSET]
        rbuf = rows[b_j % NBUF]

        @pl.loop(0, C, unroll=4)
        def _scale(e):
            wsplat = plsc.load_gather(
                wrow, [jnp.full((LANES,), e, jnp.int32)])
            row = rbuf.at[e]
            for g in range(D // LANES):
                sl = pl.ds(g * LANES, LANES)
                row[sl] = row[sl] * wsplat

    wait_idx(0)
    wait_idx(2)
    wait_idx(3)
    pltpu.async_copy(rows[1], acc.at[dst_b[2]], ssems[1], add=True)
    pltpu.async_copy(rows[2], acc.at[dst_b[3]], ssems[2], add=True)
    gather_start(0, 0)

    def chunk_step(t, j, first_group):
        scatter_wait(j - 2, j - 2)
        if not (first_group and j < 2):
            start_idx(t + 2, j + 2)
        if not (first_group and j in (1, 2)):
            wait_idx(j + 1)
        gather_start(j + 1, j + 1)
        gather_wait(j, j)
        scale(j, j)
        scatter_start(j, j)

    for j in range(GROUP):
        chunk_step(base_chunk + j, j, True)

    @pl.loop(GROUP, ROWS_W, step=GROUP)
    def _group(p):
        for j in range(GROUP):
            chunk_step(base_chunk + p + j, j, False)

    scatter_wait(ROWS_W - 2, ROWS_W - 2)
    scatter_wait(ROWS_W - 1, ROWS_W - 1)
    gather_wait(ROWS_W, ROWS_W)
    wait_idx(ROWS_W + 1)

    plsc.subcore_barrier()

    pltpu.sync_copy(acc.at[pl.ds(base_row, ROWS_PER_TILE)],
                    out_hbm.at[c, pl.ds(base_row, ROWS_PER_TILE)])


_spmm = pl.kernel(
    _spmm_body,
    out_type=jax.ShapeDtypeStruct((2, N, D), jnp.float32),
    mesh=plsc.VectorSubcoreMesh(core_axis_name="c", subcore_axis_name="s"),
    compiler_params=pltpu.CompilerParams(
        use_tc_tiling_on_sc=False, needs_layout_passes=False),
    scratch_types=(
        [pltpu.VMEM((C,), jnp.int32) for _ in range(NSET)]
        + [pltpu.VMEM((C,), jnp.int32) for _ in range(NSET)]
        + [pltpu.VMEM((C,), jnp.float32) for _ in range(NSET)]
        + [pltpu.SemaphoreType.DMA for _ in range(NSET)]
        + [pltpu.VMEM((C, D), jnp.float32) for _ in range(NBUF)]
        + [pltpu.SemaphoreType.DMA for _ in range(NBUF)]
        + [pltpu.SemaphoreType.DMA for _ in range(NBUF)]
        + [pltpu.VMEM_SHARED((N, D), jnp.float32)]
    ),
)


def kernel(x, edge_index, edge_weight, W, b):
    npad = NCHUNK_PAD * C - E
    pad_idx = jnp.arange(npad, dtype=jnp.int32) % N
    src = jnp.concatenate([edge_index[0], pad_idx]).reshape(NCHUNK_PAD, C)
    dst = jnp.concatenate([edge_index[1], pad_idx]).reshape(NCHUNK_PAD, C)
    ew = jnp.concatenate(
        [edge_weight, jnp.zeros((npad,), jnp.float32)]).reshape(NCHUNK_PAD, C)

    partials = _spmm(x, src, dst, ew)

    RB = 1000
    out = pl.pallas_call(
        _finish_body,
        grid=(N // RB,),
        in_specs=[
            pl.BlockSpec((RB, D), lambda i: (i, 0)),
            pl.BlockSpec((RB, D), lambda i: (i, 0)),
            pl.BlockSpec((D, D), lambda i: (0, 0)),
            pl.BlockSpec((1, D), lambda i: (0, 0)),
        ],
        out_specs=pl.BlockSpec((RB, D), lambda i: (i, 0)),
        out_shape=jax.ShapeDtypeStruct((N, D), jnp.float32),
    )(partials[0], partials[1], W, b.reshape(1, D))
    return out
